# Initial kernel scaffold; baseline (speedup 1.0000x reference)
import jax
import jax.numpy as jnp
from jax.experimental import pallas as pl


def kernel(x, edge_index, W0, b0, W1, b1, W2, b2):
    raise NotImplementedError("write your pallas kernel here")


# speedup vs baseline: 24.6041x; 24.6041x over previous
import jax
import jax.numpy as jnp
from jax import lax
from jax.experimental import pallas as pl
from jax.experimental.pallas import tpu as pltpu
from jax.experimental.pallas import tpu_sc as plsc

N = 50000
E = 800000
NS = 50048
EP = 819200
NCH = EP // 128
RPT = NS // 16
CH = 8

_mesh = plsc.VectorSubcoreMesh(core_axis_name="c", subcore_axis_name="s")
_params = pltpu.CompilerParams(use_tc_tiling_on_sc=False)


def _deg_kernel(dst_hbm, ones_hbm, zeros_hbm, out_hbm, dst_v, ones_v, acc_sh):
    c = lax.axis_index("c")
    s = lax.axis_index("s")
    w = c * 16 + s
    pltpu.sync_copy(ones_hbm, ones_v)
    pltpu.sync_copy(zeros_hbm, acc_sh.at[pl.ds(s * RPT, RPT)])
    plsc.subcore_barrier()
    base = w * (NCH // 32)

    def body(b, carry):
        row0 = pl.multiple_of(base + b * CH, 8)
        pltpu.sync_copy(dst_hbm.at[pl.ds(row0, CH)], dst_v)
        for j in range(CH):
            pltpu.sync_copy(ones_v, acc_sh.at[dst_v.at[j]], add=True)
        return carry

    lax.fori_loop(0, NCH // 32 // CH, body, 0)
    plsc.subcore_barrier()
    off = pl.multiple_of(c * NS + s * RPT, 8)
    pltpu.sync_copy(acc_sh.at[pl.ds(s * RPT, RPT)], out_hbm.at[pl.ds(off, RPT)])


def _make_deg():
    return pl.kernel(
        _deg_kernel,
        out_type=jax.ShapeDtypeStruct((2 * NS,), jnp.float32),
        mesh=_mesh,
        compiler_params=_params,
        scratch_types=[
            pltpu.VMEM((CH, 128), jnp.int32),
            pltpu.VMEM((128,), jnp.float32),
            pltpu.VMEM_SHARED((NS,), jnp.float32),
        ],
    )


def _agg_edge_kernel(table_hbm, src_hbm, dst_hbm, zeros_hbm, out_hbm,
                     src_v, dst_v, rows_v, sem, acc_sh):
    c = lax.axis_index("c")
    s = lax.axis_index("s")
    w = c * 16 + s
    pltpu.sync_copy(zeros_hbm, acc_sh.at[pl.ds(s * RPT, RPT)])
    plsc.subcore_barrier()
    base = w * (NCH // 32)

    def body(b, carry):
        row0 = pl.multiple_of(base + b * CH, 8)
        pltpu.sync_copy(src_hbm.at[pl.ds(row0, CH)], src_v)
        pltpu.sync_copy(dst_hbm.at[pl.ds(row0, CH)], dst_v)
        cps = [pltpu.async_copy(table_hbm.at[src_v.at[j]], rows_v.at[j], sem)
               for j in range(CH)]
        for cp in cps:
            cp.wait()
        for j in range(CH):
            pltpu.sync_copy(rows_v.at[j], acc_sh.at[dst_v.at[j]], add=True)
        return carry

    lax.fori_loop(0, NCH // 32 // CH, body, 0)
    plsc.subcore_barrier()
    pltpu.sync_copy(acc_sh.at[pl.ds(s * RPT, RPT)],
                    out_hbm.at[c].at[pl.ds(s * RPT, RPT)])


def _make_agg_edge():
    return pl.kernel(
        _agg_edge_kernel,
        out_type=jax.ShapeDtypeStruct((2, NS, 16), jnp.float32),
        mesh=_mesh,
        compiler_params=_params,
        scratch_types=[
            pltpu.VMEM((CH, 128), jnp.int32),
            pltpu.VMEM((CH, 128), jnp.int32),
            pltpu.VMEM((CH, 128, 16), jnp.float32),
            pltpu.SemaphoreType.DMA,
            pltpu.VMEM_SHARED((NS, 16), jnp.float32),
        ],
    )


def _agg_quad_kernel(table_hbm, src_hbm, dst_hbm, zeros_hbm, out_hbm,
                     src_v, dst_v, rows_v, sem, acc_sh):
    c = lax.axis_index("c")
    s = lax.axis_index("s")
    base = s * (NCH // 16)

    for r in range(2):
        q = 2 * c + r
        src_q = src_hbm.at[q]
        pltpu.sync_copy(zeros_hbm, acc_sh.at[pl.ds(s * RPT, RPT)])
        plsc.subcore_barrier()

        def body(b, carry):
            row0 = pl.multiple_of(base + b * CH, 8)
            pltpu.sync_copy(src_q.at[pl.ds(row0, CH)], src_v)
            pltpu.sync_copy(dst_hbm.at[pl.ds(row0, CH)], dst_v)
            cps = [pltpu.async_copy(table_hbm.at[src_v.at[j]], rows_v.at[j],
                                    sem) for j in range(CH)]
            for cp in cps:
                cp.wait()
            for j in range(CH):
                pltpu.sync_copy(rows_v.at[j], acc_sh.at[dst_v.at[j]], add=True)
            return carry

        lax.fori_loop(0, NCH // 16 // CH, body, 0)
        plsc.subcore_barrier()
        pltpu.sync_copy(acc_sh.at[pl.ds(s * RPT, RPT)],
                        out_hbm.at[q].at[pl.ds(s * RPT, RPT)])


def _make_agg_quad():
    return pl.kernel(
        _agg_quad_kernel,
        out_type=jax.ShapeDtypeStruct((4, NS, 16), jnp.float32),
        mesh=_mesh,
        compiler_params=_params,
        scratch_types=[
            pltpu.VMEM((CH, 128), jnp.int32),
            pltpu.VMEM((CH, 128), jnp.int32),
            pltpu.VMEM((CH, 128, 16), jnp.float32),
            pltpu.SemaphoreType.DMA,
            pltpu.VMEM_SHARED((NS, 16), jnp.float32),
        ],
    )


_R = 3128
_G = NS // _R
_LM = NS // 128


def _dinv_body(deg_ref, dinv_ref):
    rid = (lax.broadcasted_iota(jnp.int32, (_LM, 128), 0) * 128
           + lax.broadcasted_iota(jnp.int32, (_LM, 128), 1))
    deg = deg_ref[0] + deg_ref[1] + 1.0
    dinv_ref[...] = jnp.where(rid < N, lax.rsqrt(deg), 0.0)


def _b0_body(x_ref, dinv_ref, xs_ref):
    xs_ref[...] = x_ref[...] * dinv_ref[...]


def _b1_body(agg_ref, xs_ref, dinv_ref, w_ref, b_ref, out_ref):
    dinv = dinv_ref[...]
    t = (agg_ref[0] + agg_ref[1] + xs_ref[...]) * dinv
    h = jnp.maximum(
        jnp.dot(t, w_ref[...], preferred_element_type=jnp.float32) + b_ref[...],
        0.0)
    hs = h * dinv
    for q in range(4):
        out_ref[q, :, :] = hs[:, 16 * q:16 * q + 16]


def _b2_body(agg_ref, hs_ref, dinv_ref, w1_ref, b1_ref, w2_ref, out_ref):
    dinv = dinv_ref[...]
    t = jnp.concatenate([agg_ref[q] + hs_ref[q] for q in range(4)],
                        axis=1) * dinv
    h2 = jnp.maximum(
        jnp.dot(t, w1_ref[...], preferred_element_type=jnp.float32)
        + b1_ref[...], 0.0)
    out_ref[...] = jnp.dot(h2 * dinv, w2_ref[...],
                           preferred_element_type=jnp.float32)


def _b3_body(agg_ref, m_ref, dinv_ref, b_ref, out_ref):
    out_ref[...] = (dinv_ref[...] * (agg_ref[0] + agg_ref[1] + m_ref[...])
                    + b_ref[...])


def _row_spec(d):
    return pl.BlockSpec((_R, d), lambda i: (i, 0))


def _quad_spec(n, d):
    return pl.BlockSpec((n, _R, d), lambda i: (0, i, 0))


def _full_spec(shape):
    return pl.BlockSpec(shape, lambda i: tuple(0 for _ in shape))


def kernel(x, edge_index, W0, b0, W1, b1, W2, b2):
    f32 = jnp.float32
    src = edge_index[0].astype(jnp.int32)
    dst = edge_index[1].astype(jnp.int32)
    pad_vals = N + (jnp.arange(EP - E, dtype=jnp.int32) % 48)
    srcP = jnp.concatenate([src, pad_vals]).reshape(NCH, 128)
    dstP = jnp.concatenate([dst, pad_vals]).reshape(NCH, 128)
    srcO = jnp.stack([srcP + q * NS for q in range(4)])
    x_pad = jnp.pad(x, ((0, NS - N), (0, 0)))
    zeros16 = jnp.zeros((RPT, 16), f32)
    zeros1 = jnp.zeros((RPT,), f32)
    ones128 = jnp.ones((128,), f32)
    W2p = jnp.pad(W2, ((0, 0), (0, 1)))
    b0r = b0[None, :]
    b1r = b1[None, :]
    b2r = jnp.pad(b2, (0, 1))[None, :]

    deg1d = _make_deg()(dstP, ones128, zeros1)
    deg_lm = deg1d.reshape(2, NS // 128, 128)

    dinv_lm = pl.pallas_call(
        _dinv_body,
        out_shape=jax.ShapeDtypeStruct((NS // 128, 128), f32),
    )(deg_lm)
    dinv = dinv_lm.reshape(NS)[:, None]

    xs = pl.pallas_call(
        _b0_body,
        grid=(_G,),
        in_specs=[_row_spec(16), _row_spec(1)],
        out_specs=_row_spec(16),
        out_shape=jax.ShapeDtypeStruct((NS, 16), f32),
    )(x_pad, dinv)

    agg0 = _make_agg_edge()(xs, srcP, dstP, zeros16)

    hs1 = pl.pallas_call(
        _b1_body,
        grid=(_G,),
        in_specs=[_quad_spec(2, 16), _row_spec(16), _row_spec(1),
                  _full_spec((16, 64)), _full_spec((1, 64))],
        out_specs=_quad_spec(4, 16),
        out_shape=jax.ShapeDtypeStruct((4, NS, 16), f32),
    )(agg0, xs, dinv, W0, b0r)

    agg1 = _make_agg_quad()(hs1.reshape(4 * NS, 16), srcO, dstP, zeros16)

    m16 = pl.pallas_call(
        _b2_body,
        grid=(_G,),
        in_specs=[_quad_spec(4, 16), _quad_spec(4, 16), _row_spec(1),
                  _full_spec((64, 64)), _full_spec((1, 64)),
                  _full_spec((64, 16))],
        out_specs=_row_spec(16),
        out_shape=jax.ShapeDtypeStruct((NS, 16), f32),
    )(agg1, hs1, dinv, W1, b1r, W2p)

    agg2 = _make_agg_edge()(m16, srcP, dstP, zeros16)

    out16 = pl.pallas_call(
        _b3_body,
        grid=(_G,),
        in_specs=[_quad_spec(2, 16), _row_spec(16), _row_spec(1),
                  _full_spec((1, 16))],
        out_specs=_row_spec(16),
        out_shape=jax.ShapeDtypeStruct((NS, 16), f32),
    )(agg2, m16, dinv, b2r)

    return out16[:N, :15]
